# baseline (speedup 1.0000x reference)
import functools

import jax
import jax.numpy as jnp
from jax import lax
from jax.experimental import pallas as pl
from jax.experimental.pallas import tpu as pltpu
from jax.experimental.pallas import tpu_sc as plsc

_B, _S, _D = 2, 4096, 2048
_K, _R, _TOPK, _RDIM = 64, 4, 16, 16
_OUT = 2048
_SCALING = 32.0 / _TOPK
_T = _B * _S
_KR = _K * _R

_TT = 2048
_NTILES = _T // _TT

_NC, _NS, _L = 2, 16, 16
_NW = _NC * _NS
_TPW = _T // _NW
_WORDS = _TPW * _K
_UNROLL = 4


_KP = 128


def _stage1_body(x_ref, aw_ref, wq_ref, keys_ref, z_ref, s_ref):
    xt = x_ref[...]
    xb = xt.astype(jnp.bfloat16)
    awb = aw_ref[...].astype(jnp.bfloat16)
    z = lax.dot_general(xb, awb, (((1,), (1,)), ((), ())),
                        preferred_element_type=jnp.float32)
    z_ref[...] = z.astype(jnp.bfloat16)
    q = lax.dot_general(xt, wq_ref[...], (((1,), (1,)), ((), ())),
                        preferred_element_type=jnp.float32)
    s_ref[:, pl.ds(0, _K)] = lax.dot_general(
        q, keys_ref[...], (((1,), (1,)), ((), ())),
        preferred_element_type=jnp.float32)


def _stage1(xf, a_w, wq, keys, tile0, ntiles):
    return pl.pallas_call(
        _stage1_body,
        grid=(ntiles,),
        in_specs=[
            pl.BlockSpec((_TT, _D), lambda i, t0=tile0: (i + t0, 0)),
            pl.BlockSpec((_KR, _D), lambda i: (0, 0)),
            pl.BlockSpec((_RDIM, _D), lambda i: (0, 0)),
            pl.BlockSpec((_K, _RDIM), lambda i: (0, 0)),
        ],
        out_specs=[
            pl.BlockSpec((_TT, _KR), lambda i: (i, 0)),
            pl.BlockSpec((_TT, _KP), lambda i: (i, 0)),
        ],
        out_shape=[
            jax.ShapeDtypeStruct((ntiles * _TT, _KR), jnp.bfloat16),
            jax.ShapeDtypeStruct((ntiles * _TT, _KP), jnp.float32),
        ],
    )(xf, a_w, wq, keys)


def _splat(v, lane):
    idx = jnp.full((_L,), lane, jnp.int32)
    return v.at[idx].get(mode="promise_in_bounds")


def _sort_desc(v):
    return plsc.sort_key_val(v, v, descending=True)[0]


def _router_token(s_v, g_v, t):
    s = [s_v[t, pl.ds(_L * c, _L)] for c in range(4)]
    cur = _sort_desc(s[0])
    for c in range(1, 4):
        m = jnp.maximum(cur, lax.rev(_sort_desc(s[c]), (0,)))
        cur = _sort_desc(m)
    mx = _splat(cur, 0)
    tau = _splat(cur, _L - 1)
    esort = jnp.exp(cur - mx)
    zsum = _splat(lax.cumsum(esort, axis=0), _L - 1)
    rz = 1.0 / zsum
    eq16 = (cur == tau).astype(jnp.int32)
    need = _splat(lax.cumsum(eq16, axis=0), _L - 1)
    offcnt = jnp.zeros((_L,), jnp.int32)
    for c in range(4):
        sc = s[c]
        e = jnp.exp(sc - mx) * rz
        gt = sc > tau
        eq = sc == tau
        cum = lax.cumsum(eq.astype(jnp.int32), axis=0) + offcnt
        keep = jnp.logical_or(gt, jnp.logical_and(eq, cum <= need))
        g_v[t, pl.ds(_L * c, _L)] = jnp.where(keep, e, 0.0)
        offcnt = _splat(cum, _L - 1)


def _router(scores):
    ntok = scores.shape[0]
    tpw = ntok // _NW

    def body(s_hbm, g_hbm, s_v, g_v):
        wid = lax.axis_index("s") * _NC + lax.axis_index("c")
        base = wid * tpw
        pltpu.sync_copy(s_hbm.at[pl.ds(base, tpw)], s_v)

        def group(g, carry):
            for u in range(_UNROLL):
                _router_token(s_v, g_v, g * _UNROLL + u)
            return carry

        lax.fori_loop(0, tpw // _UNROLL, group, 0)
        pltpu.sync_copy(g_v, g_hbm.at[pl.ds(base, tpw)])

    fn = functools.partial(
        pl.kernel,
        mesh=plsc.VectorSubcoreMesh(core_axis_name="c", subcore_axis_name="s"),
        out_type=jax.ShapeDtypeStruct((ntok, _KP), jnp.float32),
        scratch_types=[
            pltpu.VMEM((tpw, _KP), jnp.float32),
            pltpu.VMEM((tpw, _KP), jnp.float32),
        ],
        compiler_params=pltpu.CompilerParams(needs_layout_passes=False),
    )(body)
    return fn(scores)


def _stage2_body(z_ref, g_ref, bw_ref, o_ref):
    gate = g_ref[:, pl.ds(0, _K)].astype(jnp.bfloat16)
    expand = (lax.broadcasted_iota(jnp.int32, (_K, _KR), 1) // _R ==
              lax.broadcasted_iota(jnp.int32, (_K, _KR), 0)
              ).astype(jnp.bfloat16)
    ge = lax.dot_general(gate, expand, (((1,), (0,)), ((), ())),
                         preferred_element_type=jnp.float32)
    zg = z_ref[...] * ge.astype(jnp.bfloat16)
    bwb = bw_ref[...].astype(jnp.bfloat16)
    out = lax.dot_general(zg, bwb, (((1,), (1,)), ((), ())),
                          preferred_element_type=jnp.float32)
    o_ref[...] = out * _SCALING


def _stage2_first(z_bf, gate, b_b, tile0, ntiles):
    return pl.pallas_call(
        _stage2_body,
        grid=(ntiles,),
        in_specs=[
            pl.BlockSpec((_TT, _KR), lambda i: (i, 0)),
            pl.BlockSpec((_TT, _KP), lambda i: (i, 0)),
            pl.BlockSpec((_OUT, _KR), lambda i: (0, 0)),
        ],
        out_specs=pl.BlockSpec((_TT, _OUT), lambda i, t0=tile0: (i + t0, 0)),
        out_shape=jax.ShapeDtypeStruct((_T, _OUT), jnp.float32),
    )(z_bf, gate, b_b)


def _stage2_next(acc, z_bf, gate, b_b, tile0, ntiles):
    def body(acc_ref, z_ref, g_ref, bw_ref, o_ref):
        _stage2_body(z_ref, g_ref, bw_ref, o_ref)

    return pl.pallas_call(
        body,
        grid=(ntiles,),
        in_specs=[
            pl.BlockSpec(memory_space=pl.ANY),
            pl.BlockSpec((_TT, _KR), lambda i: (i, 0)),
            pl.BlockSpec((_TT, _KP), lambda i: (i, 0)),
            pl.BlockSpec((_OUT, _KR), lambda i: (0, 0)),
        ],
        out_specs=pl.BlockSpec((_TT, _OUT), lambda i, t0=tile0: (i + t0, 0)),
        out_shape=jax.ShapeDtypeStruct((_T, _OUT), jnp.float32),
        input_output_aliases={0: 0},
    )(acc, z_bf, gate, b_b)


_NCHUNK = 2
_CTILES = _NTILES // _NCHUNK


def kernel(x, A_w, B_w, Wq_w, keys):
    xf = x.reshape(_T, _D)
    zs = [_stage1(xf, A_w, Wq_w, keys, c * _CTILES, _CTILES)
          for c in range(_NCHUNK)]
    gates = [_router(s) for _, s in zs]
    out = _stage2_first(zs[0][0], gates[0], B_w, 0, _CTILES)
    for c in range(1, _NCHUNK):
        out = _stage2_next(out, zs[c][0], gates[c], B_w, c * _CTILES, _CTILES)
    return out.reshape(_B, _S, _OUT)


# speedup vs baseline: 1.0636x; 1.0636x over previous
import functools

import jax
import jax.numpy as jnp
from jax import lax
from jax.experimental import pallas as pl
from jax.experimental.pallas import tpu as pltpu
from jax.experimental.pallas import tpu_sc as plsc

_B, _S, _D = 2, 4096, 2048
_K, _R, _TOPK, _RDIM = 64, 4, 16, 16
_OUT = 2048
_SCALING = 32.0 / _TOPK
_T = _B * _S
_KR = _K * _R

_TT = 1024
_NTILES = _T // _TT

_NC, _NS, _L = 2, 16, 16
_NW = _NC * _NS
_TPW = _T // _NW
_WORDS = _TPW * _K
_UNROLL = 4


_KP = 128


def _stage1_body(x_ref, aw_ref, wq_ref, keys_ref, z_ref, s_ref):
    xt = x_ref[...]
    xb = xt.astype(jnp.bfloat16)
    awb = aw_ref[...].astype(jnp.bfloat16)
    z = lax.dot_general(xb, awb, (((1,), (1,)), ((), ())),
                        preferred_element_type=jnp.float32)
    z_ref[...] = z.astype(jnp.bfloat16)
    q = lax.dot_general(xt, wq_ref[...], (((1,), (1,)), ((), ())),
                        preferred_element_type=jnp.float32)
    s_ref[:, pl.ds(0, _K)] = lax.dot_general(
        q, keys_ref[...], (((1,), (0,)), ((), ())),
        preferred_element_type=jnp.float32)


def _stage1(xf, a_w, wq, keys, tile0, ntiles):
    return pl.pallas_call(
        _stage1_body,
        grid=(ntiles,),
        in_specs=[
            pl.BlockSpec((_TT, _D), lambda i, t0=tile0: (i + t0, 0)),
            pl.BlockSpec((_KR, _D), lambda i: (0, 0)),
            pl.BlockSpec((_RDIM, _D), lambda i: (0, 0)),
            pl.BlockSpec((_RDIM, _K), lambda i: (0, 0)),
        ],
        out_specs=[
            pl.BlockSpec((_TT, _KR), lambda i: (i, 0)),
            pl.BlockSpec((_TT, _KP), lambda i: (i, 0)),
        ],
        out_shape=[
            jax.ShapeDtypeStruct((ntiles * _TT, _KR), jnp.bfloat16),
            jax.ShapeDtypeStruct((ntiles * _TT, _KP), jnp.float32),
        ],
    )(xf, a_w, wq, keys)


def _splat(v, lane):
    idx = jnp.full((_L,), lane, jnp.int32)
    return v.at[idx].get(mode="promise_in_bounds")


def _sort_desc(v):
    return plsc.sort_key_val(v, v, descending=True)[0]


def _router_token(s_v, g_v, t):
    s = [s_v[t, pl.ds(_L * c, _L)] for c in range(4)]
    cur = _sort_desc(s[0])
    for c in range(1, 4):
        m = jnp.maximum(cur, lax.rev(_sort_desc(s[c]), (0,)))
        cur = _sort_desc(m)
    mx = _splat(cur, 0)
    tau = _splat(cur, _L - 1)
    esort = jnp.exp(cur - mx)
    zsum = _splat(lax.cumsum(esort, axis=0), _L - 1)
    rz = 1.0 / zsum
    eq16 = (cur == tau).astype(jnp.int32)
    need = _splat(lax.cumsum(eq16, axis=0), _L - 1)
    offcnt = jnp.zeros((_L,), jnp.int32)
    for c in range(4):
        sc = s[c]
        e = jnp.exp(sc - mx) * rz
        gt = sc > tau
        eq = sc == tau
        cum = lax.cumsum(eq.astype(jnp.int32), axis=0) + offcnt
        keep = jnp.logical_or(gt, jnp.logical_and(eq, cum <= need))
        g_v[t, pl.ds(_L * c, _L)] = jnp.where(keep, e, 0.0)
        offcnt = _splat(cum, _L - 1)


def _router(scores):
    ntok = scores.shape[0]
    tpw = ntok // _NW

    def body(s_hbm, g_hbm, s_v, g_v):
        wid = lax.axis_index("s") * _NC + lax.axis_index("c")
        base = wid * tpw
        pltpu.sync_copy(s_hbm.at[pl.ds(base, tpw)], s_v)

        def group(g, carry):
            for u in range(_UNROLL):
                _router_token(s_v, g_v, g * _UNROLL + u)
            return carry

        lax.fori_loop(0, tpw // _UNROLL, group, 0)
        pltpu.sync_copy(g_v, g_hbm.at[pl.ds(base, tpw)])

    fn = functools.partial(
        pl.kernel,
        mesh=plsc.VectorSubcoreMesh(core_axis_name="c", subcore_axis_name="s"),
        out_type=jax.ShapeDtypeStruct((ntok, _KP), jnp.float32),
        scratch_types=[
            pltpu.VMEM((tpw, _KP), jnp.float32),
            pltpu.VMEM((tpw, _KP), jnp.float32),
        ],
        compiler_params=pltpu.CompilerParams(needs_layout_passes=False),
    )(body)
    return fn(scores)


def _stage2_body(z_ref, g_ref, bw_ref, o_ref):
    gate = g_ref[:, pl.ds(0, _K)].astype(jnp.bfloat16)
    expand = (lax.broadcasted_iota(jnp.int32, (_K, _KR), 1) // _R ==
              lax.broadcasted_iota(jnp.int32, (_K, _KR), 0)
              ).astype(jnp.bfloat16)
    ge = lax.dot_general(gate, expand, (((1,), (0,)), ((), ())),
                         preferred_element_type=jnp.float32)
    zg = z_ref[...] * ge.astype(jnp.bfloat16)
    bwb = bw_ref[...].astype(jnp.bfloat16)
    out = lax.dot_general(zg, bwb, (((1,), (1,)), ((), ())),
                          preferred_element_type=jnp.float32)
    o_ref[...] = out * _SCALING


def _stage2_first(z_bf, gate, b_b, tile0, ntiles):
    return pl.pallas_call(
        _stage2_body,
        grid=(ntiles,),
        in_specs=[
            pl.BlockSpec((_TT, _KR), lambda i: (i, 0)),
            pl.BlockSpec((_TT, _KP), lambda i: (i, 0)),
            pl.BlockSpec((_OUT, _KR), lambda i: (0, 0)),
        ],
        out_specs=pl.BlockSpec((_TT, _OUT), lambda i, t0=tile0: (i + t0, 0)),
        out_shape=jax.ShapeDtypeStruct((_T, _OUT), jnp.float32),
    )(z_bf, gate, b_b)


def _stage2_next(acc, z_bf, gate, b_b, tile0, ntiles):
    def body(acc_ref, z_ref, g_ref, bw_ref, o_ref):
        _stage2_body(z_ref, g_ref, bw_ref, o_ref)

    return pl.pallas_call(
        body,
        grid=(ntiles,),
        in_specs=[
            pl.BlockSpec(memory_space=pl.ANY),
            pl.BlockSpec((_TT, _KR), lambda i: (i, 0)),
            pl.BlockSpec((_TT, _KP), lambda i: (i, 0)),
            pl.BlockSpec((_OUT, _KR), lambda i: (0, 0)),
        ],
        out_specs=pl.BlockSpec((_TT, _OUT), lambda i, t0=tile0: (i + t0, 0)),
        out_shape=jax.ShapeDtypeStruct((_T, _OUT), jnp.float32),
        input_output_aliases={0: 0},
    )(acc, z_bf, gate, b_b)


_NCHUNK = 2
_CTILES = _NTILES // _NCHUNK


def kernel(x, A_w, B_w, Wq_w, keys):
    xf = x.reshape(_T, _D)
    keys_t = keys.T
    zs = [_stage1(xf, A_w, Wq_w, keys_t, c * _CTILES, _CTILES)
          for c in range(_NCHUNK)]
    gates = [_router(s) for _, s in zs]
    out = _stage2_first(zs[0][0], gates[0], B_w, 0, _CTILES)
    for c in range(1, _NCHUNK):
        out = _stage2_next(out, zs[c][0], gates[c], B_w, c * _CTILES, _CTILES)
    return out.reshape(_B, _S, _OUT)
